# baseline (speedup 1.0000x reference)
import functools

import jax
import jax.numpy as jnp
from jax import lax
from jax.experimental import pallas as pl
from jax.experimental.pallas import tpu as pltpu
from jax.experimental.pallas import tpu_sc as plsc

_B = 64
_N = 100000
_K = 3
_J = 64
_T = 128
_RPA = 8


def _make_sc_gather():
    info = plsc.get_sparse_core_info()
    nc, ns, nl = info.num_cores, info.num_subcores, info.num_lanes
    bps = _B // nc
    jpw = _J // ns
    spw = jpw * _K
    ng = (bps * _K) // nl
    na = (_RPA * _J * _K) // nl
    nq = bps // _RPA

    mesh = plsc.VectorSubcoreMesh(core_axis_name="c", subcore_axis_name="s")

    @functools.partial(
        pl.kernel,
        mesh=mesh,
        compiler_params=pltpu.CompilerParams(needs_layout_passes=False),
        out_type=(
            jax.ShapeDtypeStruct((_B, _J * _K), jnp.float32),
            jax.ShapeDtypeStruct((nc, ns, bps, spw), jnp.float32),
        ),
        scratch_types=[
            pltpu.VMEM((_J,), jnp.int32),
            pltpu.VMEM((jpw, _K, bps, _T), jnp.float32),
            pltpu.VMEM((bps, spw), jnp.float32),
            pltpu.VMEM((ns, bps, spw), jnp.float32),
            pltpu.VMEM((_RPA, _J * _K), jnp.float32),
            pltpu.SemaphoreType.DMA,
        ],
    )
    def sc_gather(pos_hbm, idx_hbm, out_hbm, sh_hbm, idx_vm, win_v, slab_v,
                  tmp_v, asm_v, sem):
        sc = lax.axis_index("c")
        s = lax.axis_index("s")
        b0 = sc * bps
        pltpu.sync_copy(idx_hbm, idx_vm)
        offs = []
        copies = []
        for t in range(jpw):
            j = s * jpw + t
            lanes = jnp.full((nl,), j, dtype=jnp.int32)
            idx_j = jnp.max(plsc.load_gather(idx_vm, [lanes]))
            c0 = pl.multiple_of((idx_j // _T) * _T, _T)
            offs.append(idx_j - c0)
            cp = pltpu.make_async_copy(
                pos_hbm.at[:, pl.ds(b0, bps), pl.ds(c0, _T)], win_v.at[t], sem
            )
            cp.start()
            copies.append(cp)
        for cp in copies:
            cp.wait()
        for t in range(jpw):
            d = offs[t]
            for g in range(ng):
                e = lax.iota(jnp.int32, nl) + g * nl
                b_vec = e // _K
                k_vec = e % _K
                vals = plsc.load_gather(
                    win_v.at[t], [k_vec, b_vec, jnp.full((nl,), d, jnp.int32)]
                )
                plsc.store_scatter(slab_v, [b_vec, k_vec + t * _K], vals)
        pltpu.sync_copy(slab_v, sh_hbm.at[sc, s])
        plsc.subcore_barrier()

        @pl.when(s < nq)
        def _assemble():
            pltpu.sync_copy(sh_hbm.at[sc], tmp_v)
            r0 = s * _RPA
            for g in range(na):
                e = lax.iota(jnp.int32, nl) + g * nl
                r_vec = e // (_J * _K)
                col = e % (_J * _K)
                s_vec = col // spw
                u_vec = col % spw
                vals = plsc.load_gather(tmp_v, [s_vec, r0 + r_vec, u_vec])
                plsc.store_scatter(asm_v, [r_vec, col], vals)
            pltpu.sync_copy(
                asm_v, out_hbm.at[pl.ds(b0 + s * _RPA, _RPA), :]
            )

    return sc_gather


_sc_gather = _make_sc_gather()


@jax.jit
def kernel(pos, idx):
    pos_t = jnp.transpose(pos, (2, 0, 1))
    idx32 = idx.astype(jnp.int32)
    out, _ = _sc_gather(pos_t, idx32)
    return out


# speedup vs baseline: 1.1432x; 1.1432x over previous
import functools

import jax
import jax.numpy as jnp
from jax import lax
from jax.experimental import pallas as pl
from jax.experimental.pallas import tpu as pltpu
from jax.experimental.pallas import tpu_sc as plsc

_B = 64
_N = 100000
_K = 3
_J = 64
_T = 128


def _make_sc_gather():
    info = plsc.get_sparse_core_info()
    nc, ns, nl = info.num_cores, info.num_subcores, info.num_lanes
    bps = _B // nc
    jpw = _J // ns
    ng = (bps * _K) // nl

    mesh = plsc.VectorSubcoreMesh(core_axis_name="c", subcore_axis_name="s")

    @functools.partial(
        pl.kernel,
        mesh=mesh,
        compiler_params=pltpu.CompilerParams(needs_layout_passes=False),
        out_type=jax.ShapeDtypeStruct((_B, ns, jpw * _K), jnp.float32),
        scratch_types=[
            pltpu.VMEM((_J,), jnp.int32),
            pltpu.VMEM((jpw, _K, bps, _T), jnp.float32),
            pltpu.VMEM((bps, 1, jpw * _K), jnp.float32),
            pltpu.SemaphoreType.DMA((jpw,)),
            pltpu.SemaphoreType.DMA,
        ],
    )
    def sc_gather(pos_hbm, idx_hbm, out_hbm, idx_vm, win_v, slab_v, wsem, sem):
        sc = lax.axis_index("c")
        s = lax.axis_index("s")
        b0 = sc * bps
        pltpu.sync_copy(idx_hbm, idx_vm)
        offs = []
        copies = []
        for t in range(jpw):
            j = s * jpw + t
            lanes = jnp.full((nl,), j, dtype=jnp.int32)
            idx_j = jnp.max(plsc.load_gather(idx_vm, [lanes]))
            c0 = pl.multiple_of((idx_j // _T) * _T, _T)
            offs.append(idx_j - c0)
            cp = pltpu.make_async_copy(
                pos_hbm.at[:, pl.ds(b0, bps), pl.ds(c0, _T)], win_v.at[t],
                wsem.at[t],
            )
            cp.start()
            copies.append(cp)
        for t in range(jpw):
            copies[t].wait()
            d = offs[t]
            for g in range(ng):
                e = lax.iota(jnp.int32, nl) + g * nl
                b_vec = e // _K
                k_vec = e % _K
                vals = plsc.load_gather(
                    win_v.at[t], [k_vec, b_vec, jnp.full((nl,), d, jnp.int32)]
                )
                plsc.store_scatter(
                    slab_v, [b_vec, jnp.zeros((nl,), jnp.int32), k_vec + t * _K], vals
                )
        pltpu.sync_copy(slab_v, out_hbm.at[pl.ds(b0, bps), pl.ds(s, 1), :])

    return sc_gather


_sc_gather = _make_sc_gather()


@jax.jit
def kernel(pos, idx):
    pos_t = jnp.transpose(pos, (2, 0, 1))
    idx32 = idx.astype(jnp.int32)
    out3 = _sc_gather(pos_t, idx32)
    return out3.reshape(_B, _J * _K)


# speedup vs baseline: 1.1564x; 1.0116x over previous
import functools

import jax
import jax.numpy as jnp
from jax import lax
from jax.experimental import pallas as pl
from jax.experimental.pallas import tpu as pltpu
from jax.experimental.pallas import tpu_sc as plsc

_B = 64
_N = 100000
_K = 3
_J = 64
_T = 128


def _make_sc_gather():
    info = plsc.get_sparse_core_info()
    nc, ns, nl = info.num_cores, info.num_subcores, info.num_lanes
    bps = _B // nc
    jpw = _J // ns
    ng = (bps * _K) // nl

    mesh = plsc.VectorSubcoreMesh(core_axis_name="c", subcore_axis_name="s")

    @functools.partial(
        pl.kernel,
        mesh=mesh,
        compiler_params=pltpu.CompilerParams(needs_layout_passes=False),
        out_type=jax.ShapeDtypeStruct((_B, ns, jpw * _K), jnp.float32),
        scratch_types=[
            pltpu.VMEM((_J,), jnp.int32),
            pltpu.VMEM((jpw, _K, bps, _T), jnp.float32),
            pltpu.VMEM((bps, 1, jpw * _K), jnp.float32),
            pltpu.SemaphoreType.DMA,
        ],
    )
    def sc_gather(pos_hbm, idx_hbm, out_hbm, idx_vm, win_v, slab_v, sem):
        sc = lax.axis_index("c")
        s = lax.axis_index("s")
        b0 = sc * bps
        pltpu.sync_copy(idx_hbm, idx_vm)
        offs = []
        copies = []
        for t in range(jpw):
            j = s * jpw + t
            lanes = jnp.full((nl,), j, dtype=jnp.int32)
            idx_j = jnp.max(plsc.load_gather(idx_vm, [lanes]))
            c0 = pl.multiple_of((idx_j // _T) * _T, _T)
            offs.append(idx_j - c0)
            cp = pltpu.make_async_copy(
                pos_hbm.at[:, pl.ds(b0, bps), pl.ds(c0, _T)], win_v.at[t], sem
            )
            cp.start()
            copies.append(cp)
        for cp in copies:
            cp.wait()
        for t in range(jpw):
            d = offs[t]
            for g in range(ng):
                e = lax.iota(jnp.int32, nl) + g * nl
                b_vec = e // _K
                k_vec = e % _K
                vals = plsc.load_gather(
                    win_v.at[t], [k_vec, b_vec, jnp.full((nl,), d, jnp.int32)]
                )
                plsc.store_scatter(
                    slab_v, [b_vec, jnp.zeros((nl,), jnp.int32), k_vec + t * _K], vals
                )
        pltpu.sync_copy(slab_v, out_hbm.at[pl.ds(b0, bps), pl.ds(s, 1), :])

    return sc_gather


_sc_gather = _make_sc_gather()


@jax.jit
def kernel(pos, idx):
    pos_t = jnp.transpose(pos, (2, 0, 1))
    idx32 = idx.astype(jnp.int32)
    out3 = _sc_gather(pos_t, idx32)
    return out3.reshape(_B, _J * _K)
